# Initial kernel scaffold; baseline (speedup 1.0000x reference)
import jax
import jax.numpy as jnp
from jax.experimental import pallas as pl


def kernel(x, edge_index, batch, lin0_W, lin0_b, lin1_W, lin1_b, agg0_W, agg0_b, agg1_W, agg1_b, cat0_W, cat0_b, cat1_W, cat1_b, ex0_W, ex0_b, ex1_W, ex1_b, ex2_W, ex2_b, ex3_W, ex3_b, pool0_W, pool0_b, pool1_W, pool1_b, pool2_W, pool2_b):
    raise NotImplementedError("write your pallas kernel here")


# speedup vs baseline: 10.8811x; 10.8811x over previous
import functools

import jax
import jax.numpy as jnp
from jax import lax
from jax.experimental import pallas as pl
from jax.experimental.pallas import tpu as pltpu
from jax.experimental.pallas import tpu_sc as plsc

N = 10000
E = 320000
HID = 128
NSUB = 512
F2 = 2 * HID
F6 = 6 * HID

NC = 2
NS = 16
NW = NC * NS
DW = 16
EC = 80
EPW = E // NW

NN = 10240
NPT = NN // NS
SPT = NSUB // NS
PC = 64
BR = 2048

_MESH = dict(core_axis_name="c", subcore_axis_name="s")


def _ids():
    c = lax.axis_index("c")
    s = lax.axis_index("s")
    return c, s, s * NC + c


@functools.partial(
    pl.kernel,
    out_type=jax.ShapeDtypeStruct((NC, NN, DW), jnp.float32),
    mesh=plsc.VectorSubcoreMesh(**_MESH),
    scratch_types=[
        pltpu.VMEM((EC,), jnp.int32),
        pltpu.VMEM((EC, DW), jnp.float32),
        pltpu.VMEM_SHARED((NN, DW), jnp.float32),
    ],
)
def _sc_degree(col_hbm, ones_hbm, zdeg_hbm, degp_hbm, cidx, ones_v, deg_sh):
    c, s, w = _ids()
    pltpu.sync_copy(ones_hbm, ones_v)
    pltpu.sync_copy(zdeg_hbm, deg_sh.at[pl.ds(NPT * s, NPT)])
    plsc.subcore_barrier()

    def step(j, carry):
        base = w * EPW + j * EC
        pltpu.sync_copy(col_hbm.at[pl.ds(base, EC)], cidx)
        pltpu.sync_copy(ones_v, deg_sh.at[cidx], add=True)
        return carry

    lax.fori_loop(0, EPW // EC, step, 0, unroll=False)
    plsc.subcore_barrier()
    pltpu.sync_copy(deg_sh.at[pl.ds(NPT * s, NPT)],
                    degp_hbm.at[c, pl.ds(NPT * s, NPT)])


@functools.partial(
    pl.kernel,
    out_type=jax.ShapeDtypeStruct((NC, NN, HID), jnp.float32),
    mesh=plsc.VectorSubcoreMesh(**_MESH),
    scratch_types=[
        pltpu.VMEM((EC,), jnp.int32),
        pltpu.VMEM((EC,), jnp.int32),
        pltpu.VMEM((EC, HID), jnp.float32),
        pltpu.VMEM_SHARED((NN, HID), jnp.float32),
        pltpu.SemaphoreType.DMA,
    ],
)
def _sc_edge_agg(hs_hbm, row_hbm, col_hbm, zacc_hbm, accp_hbm,
                 ridx, cidx, rows_v, acc_sh, sem):
    c, s, w = _ids()
    pltpu.sync_copy(zacc_hbm, acc_sh.at[pl.ds(NPT * s, NPT)])
    plsc.subcore_barrier()

    def step(j, carry):
        base = w * EPW + j * EC
        pltpu.sync_copy(row_hbm.at[pl.ds(base, EC)], ridx)
        pltpu.sync_copy(col_hbm.at[pl.ds(base, EC)], cidx)
        pltpu.async_copy(hs_hbm.at[ridx], rows_v, sem).wait()
        pltpu.sync_copy(rows_v, acc_sh.at[cidx], add=True)
        return carry

    lax.fori_loop(0, EPW // EC, step, 0, unroll=False)
    plsc.subcore_barrier()
    pltpu.sync_copy(acc_sh.at[pl.ds(NPT * s, NPT)],
                    accp_hbm.at[c, pl.ds(NPT * s, NPT)])


@functools.partial(
    pl.kernel,
    out_type=jax.ShapeDtypeStruct((NC, 6 * NSUB, HID), jnp.float32),
    mesh=plsc.VectorSubcoreMesh(**_MESH),
    scratch_types=[
        pltpu.VMEM((PC,), jnp.int32),
        pltpu.VMEM((PC, HID), jnp.float32),
        pltpu.VMEM_SHARED((6 * NSUB, HID), jnp.float32),
    ],
)
def _sc_pool(v0, v1, v2, v3, v4, v5, b6_hbm, zpool_hbm, poolp_hbm,
             bidx, vbuf, pool_sh):
    c, s, w = _ids()
    ppt = 6 * NSUB // NS
    pltpu.sync_copy(zpool_hbm, pool_sh.at[pl.ds(ppt * s, ppt)])
    plsc.subcore_barrier()
    rpw = NN // NW
    vs = (v0, v1, v2, v3, v4, v5)

    def step(j, carry):
        base = w * rpw + j * PC
        for k in range(6):
            pltpu.sync_copy(b6_hbm.at[k, pl.ds(base, PC)], bidx)
            pltpu.sync_copy(vs[k].at[pl.ds(base, PC)], vbuf)
            pltpu.sync_copy(vbuf, pool_sh.at[bidx], add=True)
        return carry

    lax.fori_loop(0, rpw // PC, step, 0, unroll=False)
    plsc.subcore_barrier()
    pltpu.sync_copy(pool_sh.at[pl.ds(ppt * s, ppt)],
                    poolp_hbm.at[c, pl.ds(ppt * s, ppt)])


def _tc_stage1(hp, degp, linW, linb, aggW):
    def body(hp_r, degp_r, linW_r, linb_r, aggW_r, h_o, hs_o):
        h = jnp.dot(hp_r[...], linW_r[...],
                    preferred_element_type=jnp.float32) + linb_r[...]
        h2 = jnp.dot(h, aggW_r[...], preferred_element_type=jnp.float32)
        d = degp_r[...]
        dinv = lax.rsqrt(d[0, :, 0:1] + d[1, :, 0:1] + 1.0)
        h_o[...] = h
        hs_o[...] = h2 * dinv

    return pl.pallas_call(
        body,
        grid=(NN // BR,),
        in_specs=[
            pl.BlockSpec((BR, HID), lambda i: (i, 0)),
            pl.BlockSpec((NC, BR, DW), lambda i: (0, i, 0)),
            pl.BlockSpec((HID, HID), lambda i: (0, 0)),
            pl.BlockSpec((1, HID), lambda i: (0, 0)),
            pl.BlockSpec((HID, HID), lambda i: (0, 0)),
        ],
        out_specs=[pl.BlockSpec((BR, HID), lambda i: (i, 0))] * 2,
        out_shape=[jax.ShapeDtypeStruct((NN, HID), jnp.float32)] * 2,
    )(hp, degp, linW, linb, aggW)


def _tc_stage2(h, hs, accp, degp, catWt, catWb, catb, aggb):
    def body(h_r, hs_r, accp_r, degp_r, wt_r, wb_r, cb_r, ab_r, z_o):
        d = degp_r[...]
        dinv = lax.rsqrt(d[0, :, 0:1] + d[1, :, 0:1] + 1.0)
        a = accp_r[...]
        agg = dinv * (a[0] + a[1] + hs_r[...]) + ab_r[...]
        t = (jnp.dot(h_r[...], wt_r[...], preferred_element_type=jnp.float32)
             + jnp.dot(agg, wb_r[...], preferred_element_type=jnp.float32)
             + cb_r[...])
        z_o[...] = jnp.tanh(t)

    return pl.pallas_call(
        body,
        grid=(NN // BR,),
        in_specs=[
            pl.BlockSpec((BR, HID), lambda i: (i, 0)),
            pl.BlockSpec((BR, HID), lambda i: (i, 0)),
            pl.BlockSpec((NC, BR, HID), lambda i: (0, i, 0)),
            pl.BlockSpec((NC, BR, DW), lambda i: (0, i, 0)),
            pl.BlockSpec((HID, HID), lambda i: (0, 0)),
            pl.BlockSpec((HID, HID), lambda i: (0, 0)),
            pl.BlockSpec((1, HID), lambda i: (0, 0)),
            pl.BlockSpec((1, HID), lambda i: (0, 0)),
        ],
        out_specs=pl.BlockSpec((BR, HID), lambda i: (i, 0)),
        out_shape=jax.ShapeDtypeStruct((NN, HID), jnp.float32),
    )(h, hs, accp, degp, catWt, catWb, catb, aggb)


def _tc_values(x, z0, z1, pw, pb):
    def body(x_r, z0_r, z1_r, pw_r, pb_r, *v_os):
        i = pl.program_id(0)
        z0b, z1b = z0_r[...], z1_r[...]
        xc = jnp.concatenate([z0b, z1b], axis=1)
        sc = jnp.exp(jnp.tanh(
            jnp.dot(xc, pw_r[...], preferred_element_type=jnp.float32)
            + pb_r[...]))
        m = (x_r[...][:, 2:5] == 1.0).astype(jnp.float32)
        s = sc * m
        rows = i * BR + lax.broadcasted_iota(jnp.int32, (BR, 1), 0)
        valid = rows < N
        zb = (z0b, z1b)
        for k in range(3):
            for j in range(2):
                v_os[2 * k + j][...] = jnp.where(
                    valid, zb[j] * s[:, k:k + 1], 0.0)

    return pl.pallas_call(
        body,
        grid=(NN // BR,),
        in_specs=[
            pl.BlockSpec((BR, HID), lambda i: (i, 0)),
            pl.BlockSpec((BR, HID), lambda i: (i, 0)),
            pl.BlockSpec((BR, HID), lambda i: (i, 0)),
            pl.BlockSpec((F2, 3), lambda i: (0, 0)),
            pl.BlockSpec((1, 3), lambda i: (0, 0)),
        ],
        out_specs=[pl.BlockSpec((BR, HID), lambda i: (i, 0))] * 6,
        out_shape=[jax.ShapeDtypeStruct((NN, HID), jnp.float32)] * 6,
    )(x, z0, z1, pw, pb)


def _tc_mlp(poolp, w0, b0, w1, b1, w2, b2, w3, b3):
    def body(pp_r, w0_r, b0_r, w1_r, b1_r, w2_r, b2_r, w3_r, b3_r, o_r):
        p = pp_r[...]
        psum = p[0] + p[1]
        xo = jnp.concatenate(
            [psum[NSUB * k:NSUB * (k + 1)] for k in range(6)], axis=1)
        h = jnp.maximum(jnp.dot(xo, w0_r[...],
                                preferred_element_type=jnp.float32)
                        + b0_r[...], 0.0)
        h = jnp.maximum(jnp.dot(h, w1_r[...],
                                preferred_element_type=jnp.float32)
                        + b1_r[...], 0.0)
        h = jnp.maximum(jnp.dot(h, w2_r[...],
                                preferred_element_type=jnp.float32)
                        + b2_r[...], 0.0)
        o_r[...] = jnp.dot(h, w3_r[...],
                           preferred_element_type=jnp.float32) + b3_r[...]

    return pl.pallas_call(
        body,
        out_shape=jax.ShapeDtypeStruct((NSUB, 4), jnp.float32),
    )(poolp, w0, b0, w1, b1, w2, b2, w3, b3)


def kernel(x, edge_index, batch,
           lin0_W, lin0_b, lin1_W, lin1_b,
           agg0_W, agg0_b, agg1_W, agg1_b,
           cat0_W, cat0_b, cat1_W, cat1_b,
           ex0_W, ex0_b, ex1_W, ex1_b, ex2_W, ex2_b, ex3_W, ex3_b,
           pool0_W, pool0_b, pool1_W, pool1_b, pool2_W, pool2_b):
    f32 = jnp.float32
    row = edge_index[0].astype(jnp.int32)
    col = edge_index[1].astype(jnp.int32)
    ones = jnp.ones((EC, DW), f32)
    zdeg = jnp.zeros((NPT, DW), f32)
    zacc = jnp.zeros((NPT, HID), f32)
    zpool = jnp.zeros((6 * NSUB // NS, HID), f32)

    degp = _sc_degree(col, ones, zdeg)

    lins = ((lin0_W, lin0_b), (lin1_W, lin1_b))
    aggs = ((agg0_W, agg0_b), (agg1_W, agg1_b))
    cats = ((cat0_W, cat0_b), (cat1_W, cat1_b))
    hp = x
    zs = []
    for i in range(2):
        linW, linb = lins[i]
        aggW, aggb = aggs[i]
        catW, catb = cats[i]
        h, hs = _tc_stage1(hp, degp, linW, linb.reshape(1, HID), aggW)
        accp = _sc_edge_agg(hs, row, col, zacc)
        z = _tc_stage2(h, hs, accp, degp, catW[:HID], catW[HID:],
                       catb.reshape(1, HID), aggb.reshape(1, HID))
        zs.append(z)
        hp = z

    pw = jnp.concatenate([pool0_W, pool1_W, pool2_W], axis=1)
    pb = jnp.concatenate([pool0_b, pool1_b, pool2_b]).reshape(1, 3)
    batch_pad = jnp.concatenate(
        [batch.astype(jnp.int32), jnp.zeros((NN - N,), jnp.int32)])
    b6 = batch_pad[None, :] + (jnp.arange(6, dtype=jnp.int32) * NSUB)[:, None]
    vplanes = _tc_values(x, zs[0], zs[1], pw, pb)
    poolp = _sc_pool(*vplanes, b6, zpool)

    return _tc_mlp(poolp,
                   ex0_W, ex0_b.reshape(1, -1), ex1_W, ex1_b.reshape(1, -1),
                   ex2_W, ex2_b.reshape(1, -1), ex3_W, ex3_b.reshape(1, -1))


# speedup vs baseline: 20.3597x; 1.8711x over previous
import functools

import jax
import jax.numpy as jnp
from jax import lax
from jax.experimental import pallas as pl
from jax.experimental.pallas import tpu as pltpu
from jax.experimental.pallas import tpu_sc as plsc

N = 10000
E = 320000
HID = 128
NSUB = 512
F2 = 2 * HID
F6 = 6 * HID

NC = 2
NS = 16
NW = NC * NS
EPW = E // NW

NN = 10240
NPT = NN // NS
SPT = NSUB // NS
PC = 64
BR = 2048

_MESH = dict(core_axis_name="c", subcore_axis_name="s")


def _ids():
    c = lax.axis_index("c")
    s = lax.axis_index("s")
    return c, s, s * NC + c


ECL = 128
EPWP = 10240
CPW = EPWP // ECL
HC = CPW // 2


@functools.partial(
    pl.kernel,
    out_type=jax.ShapeDtypeStruct((NC, NN, HID), jnp.float32),
    mesh=plsc.VectorSubcoreMesh(**_MESH),
    scratch_types=[
        pltpu.VMEM((HC, ECL), jnp.int32),
        pltpu.VMEM((ECL, HID), jnp.float32),
        pltpu.VMEM_SHARED((NN, HID), jnp.float32),
    ],
)
def _sc_degree(ones_hbm, col2_hbm, zacc_hbm, degp_hbm, cidx2, ones_v, deg_sh):
    c, s, w = _ids()
    pltpu.sync_copy(zacc_hbm, deg_sh.at[pl.ds(NPT * s, NPT)])
    pltpu.sync_copy(ones_hbm, ones_v)
    plsc.subcore_barrier()

    def phase(p):
        base = CPW * w + p * HC
        pltpu.sync_copy(col2_hbm.at[pl.ds(base, HC)], cidx2)

        def step(k, carry):
            pltpu.sync_copy(ones_v, deg_sh.at[cidx2.at[k]], add=True)
            return carry

        lax.fori_loop(0, HC, step, 0, unroll=False)

    phase(0)
    phase(1)
    plsc.subcore_barrier()
    pltpu.sync_copy(deg_sh.at[pl.ds(NPT * s, NPT)],
                    degp_hbm.at[c, pl.ds(NPT * s, NPT)])


@functools.partial(
    pl.kernel,
    out_type=jax.ShapeDtypeStruct((NC, NN, HID), jnp.float32),
    mesh=plsc.VectorSubcoreMesh(**_MESH),
    scratch_types=[
        pltpu.VMEM((HC, ECL), jnp.int32),
        pltpu.VMEM((HC, ECL), jnp.int32),
        pltpu.VMEM((ECL, HID), jnp.float32),
        pltpu.VMEM((ECL, HID), jnp.float32),
        pltpu.VMEM_SHARED((NN, HID), jnp.float32),
        pltpu.SemaphoreType.DMA,
        pltpu.SemaphoreType.DMA,
    ],
)
def _sc_edge_agg(hs_hbm, row2_hbm, col2_hbm, zacc_hbm, accp_hbm,
                 ridx2, cidx2, rows_a, rows_b, acc_sh, ga, gb):
    c, s, w = _ids()
    pltpu.sync_copy(zacc_hbm, acc_sh.at[pl.ds(NPT * s, NPT)])
    plsc.subcore_barrier()

    def phase(p):
        base = CPW * w + p * HC
        pltpu.sync_copy(row2_hbm.at[pl.ds(base, HC)], ridx2)
        pltpu.sync_copy(col2_hbm.at[pl.ds(base, HC)], cidx2)
        pltpu.async_copy(hs_hbm.at[ridx2.at[0]], rows_a, ga)
        pltpu.async_copy(hs_hbm.at[ridx2.at[1]], rows_b, gb)

        def step(j, carry):
            k = 2 * j
            pltpu.make_async_copy(hs_hbm.at[ridx2.at[0]], rows_a, ga).wait()
            pltpu.sync_copy(rows_a, acc_sh.at[cidx2.at[k]], add=True)

            @pl.when(j + 1 < HC // 2)
            def _():
                pltpu.async_copy(hs_hbm.at[ridx2.at[k + 2]], rows_a, ga)

            pltpu.make_async_copy(hs_hbm.at[ridx2.at[1]], rows_b, gb).wait()
            pltpu.sync_copy(rows_b, acc_sh.at[cidx2.at[k + 1]], add=True)

            @pl.when(j + 1 < HC // 2)
            def _():
                pltpu.async_copy(hs_hbm.at[ridx2.at[k + 3]], rows_b, gb)

            return carry

        lax.fori_loop(0, HC // 2, step, 0, unroll=False)

    phase(0)
    phase(1)
    plsc.subcore_barrier()
    pltpu.sync_copy(acc_sh.at[pl.ds(NPT * s, NPT)],
                    accp_hbm.at[c, pl.ds(NPT * s, NPT)])


@functools.partial(
    pl.kernel,
    out_type=jax.ShapeDtypeStruct((NC, 6 * NSUB, HID), jnp.float32),
    mesh=plsc.VectorSubcoreMesh(**_MESH),
    scratch_types=[
        pltpu.VMEM((PC,), jnp.int32),
        pltpu.VMEM((PC, HID), jnp.float32),
        pltpu.VMEM_SHARED((6 * NSUB, HID), jnp.float32),
    ],
)
def _sc_pool(v0, v1, v2, v3, v4, v5, b6_hbm, zpool_hbm, poolp_hbm,
             bidx, vbuf, pool_sh):
    c, s, w = _ids()
    ppt = 6 * NSUB // NS
    pltpu.sync_copy(zpool_hbm, pool_sh.at[pl.ds(ppt * s, ppt)])
    plsc.subcore_barrier()
    rpw = NN // NW
    vs = (v0, v1, v2, v3, v4, v5)

    def step(j, carry):
        base = w * rpw + j * PC
        for k in range(6):
            pltpu.sync_copy(b6_hbm.at[k, pl.ds(base, PC)], bidx)
            pltpu.sync_copy(vs[k].at[pl.ds(base, PC)], vbuf)
            pltpu.sync_copy(vbuf, pool_sh.at[bidx], add=True)
        return carry

    lax.fori_loop(0, rpw // PC, step, 0, unroll=False)
    plsc.subcore_barrier()
    pltpu.sync_copy(pool_sh.at[pl.ds(ppt * s, ppt)],
                    poolp_hbm.at[c, pl.ds(ppt * s, ppt)])


def _tc_stage1(hp, degp, linW, linb, aggW):
    def body(hp_r, degp_r, linW_r, linb_r, aggW_r, h_o, hs_o):
        h = jnp.dot(hp_r[...], linW_r[...],
                    preferred_element_type=jnp.float32) + linb_r[...]
        h2 = jnp.dot(h, aggW_r[...], preferred_element_type=jnp.float32)
        d = degp_r[...]
        dinv = lax.rsqrt(d[0, :, 0:1] + d[1, :, 0:1] + 1.0)
        h_o[...] = h
        hs_o[...] = h2 * dinv

    return pl.pallas_call(
        body,
        grid=(NN // BR,),
        in_specs=[
            pl.BlockSpec((BR, HID), lambda i: (i, 0)),
            pl.BlockSpec((NC, BR, HID), lambda i: (0, i, 0)),
            pl.BlockSpec((HID, HID), lambda i: (0, 0)),
            pl.BlockSpec((1, HID), lambda i: (0, 0)),
            pl.BlockSpec((HID, HID), lambda i: (0, 0)),
        ],
        out_specs=[pl.BlockSpec((BR, HID), lambda i: (i, 0))] * 2,
        out_shape=[jax.ShapeDtypeStruct((NN, HID), jnp.float32)] * 2,
    )(hp, degp, linW, linb, aggW)


def _tc_stage2(h, hs, accp, degp, catWt, catWb, catb, aggb):
    def body(h_r, hs_r, accp_r, degp_r, wt_r, wb_r, cb_r, ab_r, z_o):
        d = degp_r[...]
        dinv = lax.rsqrt(d[0, :, 0:1] + d[1, :, 0:1] + 1.0)
        a = accp_r[...]
        agg = dinv * (a[0] + a[1] + hs_r[...]) + ab_r[...]
        t = (jnp.dot(h_r[...], wt_r[...], preferred_element_type=jnp.float32)
             + jnp.dot(agg, wb_r[...], preferred_element_type=jnp.float32)
             + cb_r[...])
        z_o[...] = jnp.tanh(t)

    return pl.pallas_call(
        body,
        grid=(NN // BR,),
        in_specs=[
            pl.BlockSpec((BR, HID), lambda i: (i, 0)),
            pl.BlockSpec((BR, HID), lambda i: (i, 0)),
            pl.BlockSpec((NC, BR, HID), lambda i: (0, i, 0)),
            pl.BlockSpec((NC, BR, HID), lambda i: (0, i, 0)),
            pl.BlockSpec((HID, HID), lambda i: (0, 0)),
            pl.BlockSpec((HID, HID), lambda i: (0, 0)),
            pl.BlockSpec((1, HID), lambda i: (0, 0)),
            pl.BlockSpec((1, HID), lambda i: (0, 0)),
        ],
        out_specs=pl.BlockSpec((BR, HID), lambda i: (i, 0)),
        out_shape=jax.ShapeDtypeStruct((NN, HID), jnp.float32),
    )(h, hs, accp, degp, catWt, catWb, catb, aggb)


def _tc_values(x, z0, z1, pw, pb):
    def body(x_r, z0_r, z1_r, pw_r, pb_r, *v_os):
        i = pl.program_id(0)
        z0b, z1b = z0_r[...], z1_r[...]
        xc = jnp.concatenate([z0b, z1b], axis=1)
        sc = jnp.exp(jnp.tanh(
            jnp.dot(xc, pw_r[...], preferred_element_type=jnp.float32)
            + pb_r[...]))
        m = (x_r[...][:, 2:5] == 1.0).astype(jnp.float32)
        s = sc * m
        rows = i * BR + lax.broadcasted_iota(jnp.int32, (BR, 1), 0)
        valid = rows < N
        zb = (z0b, z1b)
        for k in range(3):
            for j in range(2):
                v_os[2 * k + j][...] = jnp.where(
                    valid, zb[j] * s[:, k:k + 1], 0.0)

    return pl.pallas_call(
        body,
        grid=(NN // BR,),
        in_specs=[
            pl.BlockSpec((BR, HID), lambda i: (i, 0)),
            pl.BlockSpec((BR, HID), lambda i: (i, 0)),
            pl.BlockSpec((BR, HID), lambda i: (i, 0)),
            pl.BlockSpec((F2, 3), lambda i: (0, 0)),
            pl.BlockSpec((1, 3), lambda i: (0, 0)),
        ],
        out_specs=[pl.BlockSpec((BR, HID), lambda i: (i, 0))] * 6,
        out_shape=[jax.ShapeDtypeStruct((NN, HID), jnp.float32)] * 6,
    )(x, z0, z1, pw, pb)


def _tc_mlp(poolp, w0, b0, w1, b1, w2, b2, w3, b3):
    def body(pp_r, w0_r, b0_r, w1_r, b1_r, w2_r, b2_r, w3_r, b3_r, o_r):
        p = pp_r[...]
        psum = p[0] + p[1]
        xo = jnp.concatenate(
            [psum[NSUB * k:NSUB * (k + 1)] for k in range(6)], axis=1)
        h = jnp.maximum(jnp.dot(xo, w0_r[...],
                                preferred_element_type=jnp.float32)
                        + b0_r[...], 0.0)
        h = jnp.maximum(jnp.dot(h, w1_r[...],
                                preferred_element_type=jnp.float32)
                        + b1_r[...], 0.0)
        h = jnp.maximum(jnp.dot(h, w2_r[...],
                                preferred_element_type=jnp.float32)
                        + b2_r[...], 0.0)
        o_r[...] = jnp.dot(h, w3_r[...],
                           preferred_element_type=jnp.float32) + b3_r[...]

    return pl.pallas_call(
        body,
        out_shape=jax.ShapeDtypeStruct((NSUB, 4), jnp.float32),
    )(poolp, w0, b0, w1, b1, w2, b2, w3, b3)


def kernel(x, edge_index, batch,
           lin0_W, lin0_b, lin1_W, lin1_b,
           agg0_W, agg0_b, agg1_W, agg1_b,
           cat0_W, cat0_b, cat1_W, cat1_b,
           ex0_W, ex0_b, ex1_W, ex1_b, ex2_W, ex2_b, ex3_W, ex3_b,
           pool0_W, pool0_b, pool1_W, pool1_b, pool2_W, pool2_b):
    f32 = jnp.float32
    row = edge_index[0].astype(jnp.int32)
    col = edge_index[1].astype(jnp.int32)
    npad_e = EPWP - EPW
    prow = (jnp.arange(npad_e, dtype=jnp.int32) * 41) % N
    pcol = N + jnp.arange(npad_e, dtype=jnp.int32) % (NN - N)
    row2 = jnp.concatenate(
        [row.reshape(NW, EPW), jnp.broadcast_to(prow, (NW, npad_e))],
        axis=1).reshape(NW * CPW, ECL)
    col2 = jnp.concatenate(
        [col.reshape(NW, EPW), jnp.broadcast_to(pcol, (NW, npad_e))],
        axis=1).reshape(NW * CPW, ECL)
    zacc = jnp.zeros((NPT, HID), f32)
    zpool = jnp.zeros((6 * NSUB // NS, HID), f32)

    ones = jnp.ones((ECL, HID), f32)
    degp = _sc_degree(ones, col2, zacc)

    lins = ((lin0_W, lin0_b), (lin1_W, lin1_b))
    aggs = ((agg0_W, agg0_b), (agg1_W, agg1_b))
    cats = ((cat0_W, cat0_b), (cat1_W, cat1_b))
    hp = x
    zs = []
    for i in range(2):
        linW, linb = lins[i]
        aggW, aggb = aggs[i]
        catW, catb = cats[i]
        h, hs = _tc_stage1(hp, degp, linW, linb.reshape(1, HID), aggW)
        accp = _sc_edge_agg(hs, row2, col2, zacc)
        z = _tc_stage2(h, hs, accp, degp, catW[:HID], catW[HID:],
                       catb.reshape(1, HID), aggb.reshape(1, HID))
        zs.append(z)
        hp = z

    pw = jnp.concatenate([pool0_W, pool1_W, pool2_W], axis=1)
    pb = jnp.concatenate([pool0_b, pool1_b, pool2_b]).reshape(1, 3)
    batch_pad = jnp.concatenate(
        [batch.astype(jnp.int32), jnp.zeros((NN - N,), jnp.int32)])
    b6 = batch_pad[None, :] + (jnp.arange(8, dtype=jnp.int32) * NSUB)[:, None]
    vplanes = _tc_values(x, zs[0], zs[1], pw, pb)
    poolp = _sc_pool(*vplanes, b6, zpool)

    return _tc_mlp(poolp,
                   ex0_W, ex0_b.reshape(1, -1), ex1_W, ex1_b.reshape(1, -1),
                   ex2_W, ex2_b.reshape(1, -1), ex3_W, ex3_b.reshape(1, -1))


# speedup vs baseline: 21.8030x; 1.0709x over previous
import functools

import jax
import jax.numpy as jnp
from jax import lax
from jax.experimental import pallas as pl
from jax.experimental.pallas import tpu as pltpu
from jax.experimental.pallas import tpu_sc as plsc

N = 10000
E = 320000
HID = 128
NSUB = 512
F2 = 2 * HID
F6 = 6 * HID

NC = 2
NS = 16
NW = NC * NS
EPW = E // NW

NN = 10240
NPT = NN // NS
SPT = NSUB // NS
PC = 64
BR = 2048

_MESH = dict(core_axis_name="c", subcore_axis_name="s")


def _ids():
    c = lax.axis_index("c")
    s = lax.axis_index("s")
    return c, s, s * NC + c


ECL = 128
EPWP = 10240
CPW = EPWP // ECL
HC = CPW // 2


@functools.partial(
    pl.kernel,
    out_type=jax.ShapeDtypeStruct((NC, NN, HID), jnp.float32),
    mesh=plsc.VectorSubcoreMesh(**_MESH),
    scratch_types=[
        pltpu.VMEM((HC, ECL), jnp.int32),
        pltpu.VMEM((ECL, HID), jnp.float32),
        pltpu.VMEM_SHARED((NN, HID), jnp.float32),
    ],
)
def _sc_degree(ones_hbm, col2_hbm, zacc_hbm, degp_hbm, cidx2, ones_v, deg_sh):
    c, s, w = _ids()
    pltpu.sync_copy(zacc_hbm, deg_sh.at[pl.ds(NPT * s, NPT)])
    pltpu.sync_copy(ones_hbm, ones_v)
    plsc.subcore_barrier()

    def phase(p):
        base = CPW * w + p * HC
        pltpu.sync_copy(col2_hbm.at[pl.ds(base, HC)], cidx2)

        def step(k, carry):
            pltpu.sync_copy(ones_v, deg_sh.at[cidx2.at[k]], add=True)
            return carry

        lax.fori_loop(0, HC, step, 0, unroll=False)

    phase(0)
    phase(1)
    plsc.subcore_barrier()
    pltpu.sync_copy(deg_sh.at[pl.ds(NPT * s, NPT)],
                    degp_hbm.at[c, pl.ds(NPT * s, NPT)])


@functools.partial(
    pl.kernel,
    out_type=jax.ShapeDtypeStruct((NC, NN, HID), jnp.float32),
    mesh=plsc.VectorSubcoreMesh(**_MESH),
    scratch_types=[
        pltpu.VMEM((HC, ECL), jnp.int32),
        pltpu.VMEM((HC, ECL), jnp.int32),
        pltpu.VMEM((ECL, HID), jnp.float32),
        pltpu.VMEM((ECL, HID), jnp.float32),
        pltpu.VMEM_SHARED((NN, HID), jnp.float32),
        pltpu.SemaphoreType.DMA,
        pltpu.SemaphoreType.DMA,
    ],
)
def _sc_edge_agg(hs_hbm, row2_hbm, col2_hbm, zacc_hbm, accp_hbm,
                 ridx2, cidx2, rows_a, rows_b, acc_sh, ga, gb):
    c, s, w = _ids()
    pltpu.sync_copy(zacc_hbm, acc_sh.at[pl.ds(NPT * s, NPT)])
    plsc.subcore_barrier()

    def phase(p):
        base = CPW * w + p * HC
        pltpu.sync_copy(row2_hbm.at[pl.ds(base, HC)], ridx2)
        pltpu.sync_copy(col2_hbm.at[pl.ds(base, HC)], cidx2)
        pltpu.async_copy(hs_hbm.at[ridx2.at[0]], rows_a, ga)
        pltpu.async_copy(hs_hbm.at[ridx2.at[1]], rows_b, gb)

        def step(j, carry):
            k = 2 * j
            pltpu.make_async_copy(hs_hbm.at[ridx2.at[0]], rows_a, ga).wait()
            pltpu.sync_copy(rows_a, acc_sh.at[cidx2.at[k]], add=True)

            @pl.when(j + 1 < HC // 2)
            def _():
                pltpu.async_copy(hs_hbm.at[ridx2.at[k + 2]], rows_a, ga)

            pltpu.make_async_copy(hs_hbm.at[ridx2.at[1]], rows_b, gb).wait()
            pltpu.sync_copy(rows_b, acc_sh.at[cidx2.at[k + 1]], add=True)

            @pl.when(j + 1 < HC // 2)
            def _():
                pltpu.async_copy(hs_hbm.at[ridx2.at[k + 3]], rows_b, gb)

            return carry

        lax.fori_loop(0, HC // 2, step, 0, unroll=False)

    phase(0)
    phase(1)
    plsc.subcore_barrier()
    pltpu.sync_copy(acc_sh.at[pl.ds(NPT * s, NPT)],
                    accp_hbm.at[c, pl.ds(NPT * s, NPT)])


NSTEP = 6 * (NN // NW) // PC


@functools.partial(
    pl.kernel,
    out_type=jax.ShapeDtypeStruct((NC, 6 * NSUB, HID), jnp.float32),
    mesh=plsc.VectorSubcoreMesh(**_MESH),
    scratch_types=[
        pltpu.VMEM((PC,), jnp.int32),
        pltpu.VMEM((PC,), jnp.int32),
        pltpu.VMEM((PC, HID), jnp.float32),
        pltpu.VMEM((PC, HID), jnp.float32),
        pltpu.VMEM_SHARED((6 * NSUB, HID), jnp.float32),
        pltpu.SemaphoreType.DMA,
        pltpu.SemaphoreType.DMA,
        pltpu.SemaphoreType.DMA,
        pltpu.SemaphoreType.DMA,
    ],
)
def _sc_pool(v6_hbm, b6_hbm, zpool_hbm, poolp_hbm,
             bidx_a, bidx_b, vbuf_a, vbuf_b, pool_sh, ia, ib, va, vb):
    c, s, w = _ids()
    ppt = 6 * NSUB // NS
    pltpu.sync_copy(zpool_hbm, pool_sh.at[pl.ds(ppt * s, ppt)])
    plsc.subcore_barrier()
    rpw = NN // NW

    def load(t, bidx, vbuf, si, sv):
        k = t % 6
        base = w * rpw + (t // 6) * PC
        pltpu.async_copy(b6_hbm.at[k, pl.ds(base, PC)], bidx, si)
        pltpu.async_copy(v6_hbm.at[k, pl.ds(base, PC)], vbuf, sv)

    def drain(bidx, vbuf, si, sv):
        pltpu.make_async_copy(b6_hbm.at[0, pl.ds(0, PC)], bidx, si).wait()
        pltpu.make_async_copy(v6_hbm.at[0, pl.ds(0, PC)], vbuf, sv).wait()

    load(0, bidx_a, vbuf_a, ia, va)
    load(1, bidx_b, vbuf_b, ib, vb)

    def step(j, carry):
        t = 2 * j
        drain(bidx_a, vbuf_a, ia, va)
        pltpu.sync_copy(vbuf_a, pool_sh.at[bidx_a], add=True)

        @pl.when(j + 1 < NSTEP // 2)
        def _():
            load(t + 2, bidx_a, vbuf_a, ia, va)

        drain(bidx_b, vbuf_b, ib, vb)
        pltpu.sync_copy(vbuf_b, pool_sh.at[bidx_b], add=True)

        @pl.when(j + 1 < NSTEP // 2)
        def _():
            load(t + 3, bidx_b, vbuf_b, ib, vb)

        return carry

    lax.fori_loop(0, NSTEP // 2, step, 0, unroll=False)
    plsc.subcore_barrier()
    pltpu.sync_copy(pool_sh.at[pl.ds(ppt * s, ppt)],
                    poolp_hbm.at[c, pl.ds(ppt * s, ppt)])


def _tc_stage1(hp, degp, linW, linb, aggW):
    def body(hp_r, degp_r, linW_r, linb_r, aggW_r, h_o, hs_o):
        h = jnp.dot(hp_r[...], linW_r[...],
                    preferred_element_type=jnp.float32) + linb_r[...]
        h2 = jnp.dot(h, aggW_r[...], preferred_element_type=jnp.float32)
        d = degp_r[...]
        dinv = lax.rsqrt(d[0, :, 0:1] + d[1, :, 0:1] + 1.0)
        h_o[...] = h
        hs_o[...] = h2 * dinv

    return pl.pallas_call(
        body,
        grid=(NN // BR,),
        in_specs=[
            pl.BlockSpec((BR, HID), lambda i: (i, 0)),
            pl.BlockSpec((NC, BR, HID), lambda i: (0, i, 0)),
            pl.BlockSpec((HID, HID), lambda i: (0, 0)),
            pl.BlockSpec((1, HID), lambda i: (0, 0)),
            pl.BlockSpec((HID, HID), lambda i: (0, 0)),
        ],
        out_specs=[pl.BlockSpec((BR, HID), lambda i: (i, 0))] * 2,
        out_shape=[jax.ShapeDtypeStruct((NN, HID), jnp.float32)] * 2,
    )(hp, degp, linW, linb, aggW)


def _tc_stage2(h, hs, accp, degp, catWt, catWb, catb, aggb):
    def body(h_r, hs_r, accp_r, degp_r, wt_r, wb_r, cb_r, ab_r, z_o):
        d = degp_r[...]
        dinv = lax.rsqrt(d[0, :, 0:1] + d[1, :, 0:1] + 1.0)
        a = accp_r[...]
        agg = dinv * (a[0] + a[1] + hs_r[...]) + ab_r[...]
        t = (jnp.dot(h_r[...], wt_r[...], preferred_element_type=jnp.float32)
             + jnp.dot(agg, wb_r[...], preferred_element_type=jnp.float32)
             + cb_r[...])
        z_o[...] = jnp.tanh(t)

    return pl.pallas_call(
        body,
        grid=(NN // BR,),
        in_specs=[
            pl.BlockSpec((BR, HID), lambda i: (i, 0)),
            pl.BlockSpec((BR, HID), lambda i: (i, 0)),
            pl.BlockSpec((NC, BR, HID), lambda i: (0, i, 0)),
            pl.BlockSpec((NC, BR, HID), lambda i: (0, i, 0)),
            pl.BlockSpec((HID, HID), lambda i: (0, 0)),
            pl.BlockSpec((HID, HID), lambda i: (0, 0)),
            pl.BlockSpec((1, HID), lambda i: (0, 0)),
            pl.BlockSpec((1, HID), lambda i: (0, 0)),
        ],
        out_specs=pl.BlockSpec((BR, HID), lambda i: (i, 0)),
        out_shape=jax.ShapeDtypeStruct((NN, HID), jnp.float32),
    )(h, hs, accp, degp, catWt, catWb, catb, aggb)


def _tc_values(x, z0, z1, pw, pb):
    def body(x_r, z0_r, z1_r, pw_r, pb_r, *v_os):
        i = pl.program_id(0)
        z0b, z1b = z0_r[...], z1_r[...]
        xc = jnp.concatenate([z0b, z1b], axis=1)
        sc = jnp.exp(jnp.tanh(
            jnp.dot(xc, pw_r[...], preferred_element_type=jnp.float32)
            + pb_r[...]))
        m = (x_r[...][:, 2:5] == 1.0).astype(jnp.float32)
        s = sc * m
        rows = i * BR + lax.broadcasted_iota(jnp.int32, (BR, 1), 0)
        valid = rows < N
        zb = (z0b, z1b)
        v_o = v_os[0]
        for k in range(3):
            for j in range(2):
                v_o[2 * k + j] = jnp.where(
                    valid, zb[j] * s[:, k:k + 1], 0.0)

    return pl.pallas_call(
        body,
        grid=(NN // BR,),
        in_specs=[
            pl.BlockSpec((BR, HID), lambda i: (i, 0)),
            pl.BlockSpec((BR, HID), lambda i: (i, 0)),
            pl.BlockSpec((BR, HID), lambda i: (i, 0)),
            pl.BlockSpec((F2, 3), lambda i: (0, 0)),
            pl.BlockSpec((1, 3), lambda i: (0, 0)),
        ],
        out_specs=pl.BlockSpec((6, BR, HID), lambda i: (0, i, 0)),
        out_shape=jax.ShapeDtypeStruct((6, NN, HID), jnp.float32),
    )(x, z0, z1, pw, pb)


def _tc_mlp(poolp, w0, b0, w1, b1, w2, b2, w3, b3):
    def body(pp_r, w0_r, b0_r, w1_r, b1_r, w2_r, b2_r, w3_r, b3_r, o_r):
        p = pp_r[...]
        psum = p[0] + p[1]
        xo = jnp.concatenate(
            [psum[NSUB * k:NSUB * (k + 1)] for k in range(6)], axis=1)
        h = jnp.maximum(jnp.dot(xo, w0_r[...],
                                preferred_element_type=jnp.float32)
                        + b0_r[...], 0.0)
        h = jnp.maximum(jnp.dot(h, w1_r[...],
                                preferred_element_type=jnp.float32)
                        + b1_r[...], 0.0)
        h = jnp.maximum(jnp.dot(h, w2_r[...],
                                preferred_element_type=jnp.float32)
                        + b2_r[...], 0.0)
        o_r[...] = jnp.dot(h, w3_r[...],
                           preferred_element_type=jnp.float32) + b3_r[...]

    return pl.pallas_call(
        body,
        out_shape=jax.ShapeDtypeStruct((NSUB, 4), jnp.float32),
    )(poolp, w0, b0, w1, b1, w2, b2, w3, b3)


def kernel(x, edge_index, batch,
           lin0_W, lin0_b, lin1_W, lin1_b,
           agg0_W, agg0_b, agg1_W, agg1_b,
           cat0_W, cat0_b, cat1_W, cat1_b,
           ex0_W, ex0_b, ex1_W, ex1_b, ex2_W, ex2_b, ex3_W, ex3_b,
           pool0_W, pool0_b, pool1_W, pool1_b, pool2_W, pool2_b):
    f32 = jnp.float32
    row = edge_index[0].astype(jnp.int32)
    col = edge_index[1].astype(jnp.int32)
    npad_e = EPWP - EPW
    prow = (jnp.arange(npad_e, dtype=jnp.int32) * 41) % N
    pcol = N + jnp.arange(npad_e, dtype=jnp.int32) % (NN - N)
    row2 = jnp.concatenate(
        [row.reshape(NW, EPW), jnp.broadcast_to(prow, (NW, npad_e))],
        axis=1).reshape(NW * CPW, ECL)
    col2 = jnp.concatenate(
        [col.reshape(NW, EPW), jnp.broadcast_to(pcol, (NW, npad_e))],
        axis=1).reshape(NW * CPW, ECL)
    zacc = jnp.zeros((NPT, HID), f32)
    zpool = jnp.zeros((6 * NSUB // NS, HID), f32)

    ones = jnp.ones((ECL, HID), f32)
    degp = _sc_degree(ones, col2, zacc)

    lins = ((lin0_W, lin0_b), (lin1_W, lin1_b))
    aggs = ((agg0_W, agg0_b), (agg1_W, agg1_b))
    cats = ((cat0_W, cat0_b), (cat1_W, cat1_b))
    hp = x
    zs = []
    for i in range(2):
        linW, linb = lins[i]
        aggW, aggb = aggs[i]
        catW, catb = cats[i]
        h, hs = _tc_stage1(hp, degp, linW, linb.reshape(1, HID), aggW)
        accp = _sc_edge_agg(hs, row2, col2, zacc)
        z = _tc_stage2(h, hs, accp, degp, catW[:HID], catW[HID:],
                       catb.reshape(1, HID), aggb.reshape(1, HID))
        zs.append(z)
        hp = z

    pw = jnp.concatenate([pool0_W, pool1_W, pool2_W], axis=1)
    pb = jnp.concatenate([pool0_b, pool1_b, pool2_b]).reshape(1, 3)
    batch_pad = jnp.concatenate(
        [batch.astype(jnp.int32), jnp.zeros((NN - N,), jnp.int32)])
    b6 = batch_pad[None, :] + (jnp.arange(8, dtype=jnp.int32) * NSUB)[:, None]
    v6 = _tc_values(x, zs[0], zs[1], pw, pb)
    poolp = _sc_pool(v6, b6, zpool)

    return _tc_mlp(poolp,
                   ex0_W, ex0_b.reshape(1, -1), ex1_W, ex1_b.reshape(1, -1),
                   ex2_W, ex2_b.reshape(1, -1), ex3_W, ex3_b.reshape(1, -1))


# speedup vs baseline: 22.5087x; 1.0324x over previous
import functools

import jax
import jax.numpy as jnp
from jax import lax
from jax.experimental import pallas as pl
from jax.experimental.pallas import tpu as pltpu
from jax.experimental.pallas import tpu_sc as plsc

N = 10000
E = 320000
HID = 128
NSUB = 512
F2 = 2 * HID
F6 = 6 * HID

NC = 2
NS = 16
NW = NC * NS
EPW = E // NW

NN = 10240
NPT = NN // NS
SPT = NSUB // NS
PC = 64
BR = 2048

_MESH = dict(core_axis_name="c", subcore_axis_name="s")


def _ids():
    c = lax.axis_index("c")
    s = lax.axis_index("s")
    return c, s, s * NC + c


ECL = 128
EPWP = 10240
CPW = EPWP // ECL
HC = CPW // 2


@functools.partial(
    pl.kernel,
    out_type=jax.ShapeDtypeStruct((NC, NN, HID), jnp.float32),
    mesh=plsc.VectorSubcoreMesh(**_MESH),
    scratch_types=[
        pltpu.VMEM((HC, ECL), jnp.int32),
        pltpu.VMEM((ECL, HID), jnp.float32),
        pltpu.VMEM_SHARED((NN, HID), jnp.float32),
    ],
)
def _sc_degree(ones_hbm, col2_hbm, zacc_hbm, degp_hbm, cidx2, ones_v, deg_sh):
    c, s, w = _ids()
    pltpu.sync_copy(zacc_hbm, deg_sh.at[pl.ds(NPT * s, NPT)])
    pltpu.sync_copy(ones_hbm, ones_v)
    plsc.subcore_barrier()

    def phase(p):
        base = CPW * w + p * HC
        pltpu.sync_copy(col2_hbm.at[pl.ds(base, HC)], cidx2)

        def step(k, carry):
            pltpu.sync_copy(ones_v, deg_sh.at[cidx2.at[k]], add=True)
            return carry

        lax.fori_loop(0, HC, step, 0, unroll=False)

    phase(0)
    phase(1)
    plsc.subcore_barrier()
    pltpu.sync_copy(deg_sh.at[pl.ds(NPT * s, NPT)],
                    degp_hbm.at[c, pl.ds(NPT * s, NPT)])


@functools.partial(
    pl.kernel,
    out_type=jax.ShapeDtypeStruct((NC, NN, HID), jnp.float32),
    mesh=plsc.VectorSubcoreMesh(**_MESH),
    scratch_types=[
        pltpu.VMEM((HC, ECL), jnp.int32),
        pltpu.VMEM((HC, ECL), jnp.int32),
        pltpu.VMEM((ECL, HID), jnp.float32),
        pltpu.VMEM((ECL, HID), jnp.float32),
        pltpu.VMEM_SHARED((NN, HID), jnp.float32),
        pltpu.SemaphoreType.DMA,
        pltpu.SemaphoreType.DMA,
    ],
)
def _sc_edge_agg(hs_hbm, row2_hbm, col2_hbm, zacc_hbm, accp_hbm,
                 ridx2, cidx2, rows_a, rows_b, acc_sh, ga, gb):
    c, s, w = _ids()
    pltpu.sync_copy(zacc_hbm, acc_sh.at[pl.ds(NPT * s, NPT)])
    plsc.subcore_barrier()

    def phase(p):
        base = CPW * w + p * HC
        pltpu.sync_copy(row2_hbm.at[pl.ds(base, HC)], ridx2)
        pltpu.sync_copy(col2_hbm.at[pl.ds(base, HC)], cidx2)
        pltpu.async_copy(hs_hbm.at[ridx2.at[0]], rows_a, ga)
        pltpu.async_copy(hs_hbm.at[ridx2.at[1]], rows_b, gb)

        def step(j, carry):
            k = 2 * j
            pltpu.make_async_copy(hs_hbm.at[ridx2.at[0]], rows_a, ga).wait()
            pltpu.sync_copy(rows_a, acc_sh.at[cidx2.at[k]], add=True)

            @pl.when(j + 1 < HC // 2)
            def _():
                pltpu.async_copy(hs_hbm.at[ridx2.at[k + 2]], rows_a, ga)

            pltpu.make_async_copy(hs_hbm.at[ridx2.at[1]], rows_b, gb).wait()
            pltpu.sync_copy(rows_b, acc_sh.at[cidx2.at[k + 1]], add=True)

            @pl.when(j + 1 < HC // 2)
            def _():
                pltpu.async_copy(hs_hbm.at[ridx2.at[k + 3]], rows_b, gb)

            return carry

        lax.fori_loop(0, HC // 2, step, 0, unroll=False)

    phase(0)
    phase(1)
    plsc.subcore_barrier()
    pltpu.sync_copy(acc_sh.at[pl.ds(NPT * s, NPT)],
                    accp_hbm.at[c, pl.ds(NPT * s, NPT)])


NSTEP = 6 * (NN // NW) // PC


@functools.partial(
    pl.kernel,
    out_type=jax.ShapeDtypeStruct((NC, 6 * NSUB, HID), jnp.float32),
    mesh=plsc.VectorSubcoreMesh(**_MESH),
    scratch_types=[
        pltpu.VMEM((PC,), jnp.int32),
        pltpu.VMEM((PC,), jnp.int32),
        pltpu.VMEM((PC, HID), jnp.float32),
        pltpu.VMEM((PC, HID), jnp.float32),
        pltpu.VMEM_SHARED((6 * NSUB, HID), jnp.float32),
        pltpu.SemaphoreType.DMA,
        pltpu.SemaphoreType.DMA,
        pltpu.SemaphoreType.DMA,
        pltpu.SemaphoreType.DMA,
    ],
)
def _sc_pool(v6_hbm, b6_hbm, zpool_hbm, poolp_hbm,
             bidx_a, bidx_b, vbuf_a, vbuf_b, pool_sh, ia, ib, va, vb):
    c, s, w = _ids()
    ppt = 6 * NSUB // NS
    pltpu.sync_copy(zpool_hbm, pool_sh.at[pl.ds(ppt * s, ppt)])
    plsc.subcore_barrier()
    rpw = NN // NW

    def load(t, bidx, vbuf, si, sv):
        k = t % 6
        base = w * rpw + (t // 6) * PC
        pltpu.async_copy(b6_hbm.at[k, pl.ds(base, PC)], bidx, si)
        pltpu.async_copy(v6_hbm.at[k, pl.ds(base, PC)], vbuf, sv)

    def drain(bidx, vbuf, si, sv):
        pltpu.make_async_copy(b6_hbm.at[0, pl.ds(0, PC)], bidx, si).wait()
        pltpu.make_async_copy(v6_hbm.at[0, pl.ds(0, PC)], vbuf, sv).wait()

    load(0, bidx_a, vbuf_a, ia, va)
    load(1, bidx_b, vbuf_b, ib, vb)

    def step(j, carry):
        t = 2 * j
        drain(bidx_a, vbuf_a, ia, va)
        pltpu.sync_copy(vbuf_a, pool_sh.at[bidx_a], add=True)

        @pl.when(j + 1 < NSTEP // 2)
        def _():
            load(t + 2, bidx_a, vbuf_a, ia, va)

        drain(bidx_b, vbuf_b, ib, vb)
        pltpu.sync_copy(vbuf_b, pool_sh.at[bidx_b], add=True)

        @pl.when(j + 1 < NSTEP // 2)
        def _():
            load(t + 3, bidx_b, vbuf_b, ib, vb)

        return carry

    lax.fori_loop(0, NSTEP // 2, step, 0, unroll=False)
    plsc.subcore_barrier()
    pltpu.sync_copy(pool_sh.at[pl.ds(ppt * s, ppt)],
                    poolp_hbm.at[c, pl.ds(ppt * s, ppt)])


def _tc_stage1(hp, degp, linW, linb, aggW):
    def body(hp_r, degp_r, linW_r, linb_r, aggW_r, h_o, hs_o):
        h = jnp.dot(hp_r[...], linW_r[...],
                    preferred_element_type=jnp.float32) + linb_r[...]
        h2 = jnp.dot(h, aggW_r[...], preferred_element_type=jnp.float32)
        d = degp_r[...].astype(jnp.float32)
        dinv = lax.rsqrt(d[0, :, 0:1] + d[1, :, 0:1] + 1.0)
        h_o[...] = h
        hs_o[...] = h2 * dinv

    return pl.pallas_call(
        body,
        grid=(NN // BR,),
        in_specs=[
            pl.BlockSpec((BR, HID), lambda i: (i, 0)),
            pl.BlockSpec((NC, BR, HID), lambda i: (0, i, 0)),
            pl.BlockSpec((HID, HID), lambda i: (0, 0)),
            pl.BlockSpec((1, HID), lambda i: (0, 0)),
            pl.BlockSpec((HID, HID), lambda i: (0, 0)),
        ],
        out_specs=[pl.BlockSpec((BR, HID), lambda i: (i, 0))] * 2,
        out_shape=[jax.ShapeDtypeStruct((NN, HID), jnp.float32)] * 2,
    )(hp, degp, linW, linb, aggW)


def _tc_stage2_stage1(h, hs, accp, degp, catWt, catWb, catb, aggb,
                      linW, linb, aggW):
    def body(h_r, hs_r, accp_r, degp_r, wt_r, wb_r, cb_r, ab_r,
             linW_r, linb_r, aggW_r, z_o, h1_o, hs1_o):
        d = degp_r[...].astype(jnp.float32)
        dinv = lax.rsqrt(d[0, :, 0:1] + d[1, :, 0:1] + 1.0)
        a = accp_r[...]
        agg = dinv * (a[0] + a[1] + hs_r[...]) + ab_r[...]
        z = jnp.tanh(
            jnp.dot(h_r[...], wt_r[...], preferred_element_type=jnp.float32)
            + jnp.dot(agg, wb_r[...], preferred_element_type=jnp.float32)
            + cb_r[...])
        h1 = jnp.dot(z, linW_r[...],
                     preferred_element_type=jnp.float32) + linb_r[...]
        h21 = jnp.dot(h1, aggW_r[...], preferred_element_type=jnp.float32)
        z_o[...] = z
        h1_o[...] = h1
        hs1_o[...] = h21 * dinv

    return pl.pallas_call(
        body,
        grid=(NN // BR,),
        in_specs=[
            pl.BlockSpec((BR, HID), lambda i: (i, 0)),
            pl.BlockSpec((BR, HID), lambda i: (i, 0)),
            pl.BlockSpec((NC, BR, HID), lambda i: (0, i, 0)),
            pl.BlockSpec((NC, BR, HID), lambda i: (0, i, 0)),
            pl.BlockSpec((HID, HID), lambda i: (0, 0)),
            pl.BlockSpec((HID, HID), lambda i: (0, 0)),
            pl.BlockSpec((1, HID), lambda i: (0, 0)),
            pl.BlockSpec((1, HID), lambda i: (0, 0)),
            pl.BlockSpec((HID, HID), lambda i: (0, 0)),
            pl.BlockSpec((1, HID), lambda i: (0, 0)),
            pl.BlockSpec((HID, HID), lambda i: (0, 0)),
        ],
        out_specs=[pl.BlockSpec((BR, HID), lambda i: (i, 0))] * 3,
        out_shape=[jax.ShapeDtypeStruct((NN, HID), jnp.float32)] * 3,
    )(h, hs, accp, degp, catWt, catWb, catb, aggb, linW, linb, aggW)


def _tc_stage2_values(h, hs, accp, degp, catWt, catWb, catb, aggb,
                      x, z0, pw, pb):
    def body(h_r, hs_r, accp_r, degp_r, wt_r, wb_r, cb_r, ab_r,
             x_r, z0_r, pw_r, pb_r, *v_os):
        i = pl.program_id(0)
        d = degp_r[...].astype(jnp.float32)
        dinv = lax.rsqrt(d[0, :, 0:1] + d[1, :, 0:1] + 1.0)
        a = accp_r[...]
        agg = dinv * (a[0] + a[1] + hs_r[...]) + ab_r[...]
        z1b = jnp.tanh(
            jnp.dot(h_r[...], wt_r[...], preferred_element_type=jnp.float32)
            + jnp.dot(agg, wb_r[...], preferred_element_type=jnp.float32)
            + cb_r[...])
        z0b = z0_r[...]
        xc = jnp.concatenate([z0b, z1b], axis=1)
        sc = jnp.exp(jnp.tanh(
            jnp.dot(xc, pw_r[...], preferred_element_type=jnp.float32)
            + pb_r[...]))
        m = (x_r[...][:, 2:5] == 1.0).astype(jnp.float32)
        s = sc * m
        rows = i * BR + lax.broadcasted_iota(jnp.int32, (BR, 1), 0)
        valid = rows < N
        zb = (z0b, z1b)
        v_o = v_os[0]
        for k in range(3):
            for j in range(2):
                v_o[2 * k + j] = jnp.where(
                    valid, zb[j] * s[:, k:k + 1], 0.0)

    return pl.pallas_call(
        body,
        grid=(NN // BR,),
        in_specs=[
            pl.BlockSpec((BR, HID), lambda i: (i, 0)),
            pl.BlockSpec((BR, HID), lambda i: (i, 0)),
            pl.BlockSpec((NC, BR, HID), lambda i: (0, i, 0)),
            pl.BlockSpec((NC, BR, HID), lambda i: (0, i, 0)),
            pl.BlockSpec((HID, HID), lambda i: (0, 0)),
            pl.BlockSpec((HID, HID), lambda i: (0, 0)),
            pl.BlockSpec((1, HID), lambda i: (0, 0)),
            pl.BlockSpec((1, HID), lambda i: (0, 0)),
            pl.BlockSpec((BR, HID), lambda i: (i, 0)),
            pl.BlockSpec((BR, HID), lambda i: (i, 0)),
            pl.BlockSpec((F2, 3), lambda i: (0, 0)),
            pl.BlockSpec((1, 3), lambda i: (0, 0)),
        ],
        out_specs=pl.BlockSpec((6, BR, HID), lambda i: (0, i, 0)),
        out_shape=jax.ShapeDtypeStruct((6, NN, HID), jnp.float32),
    )(h, hs, accp, degp, catWt, catWb, catb, aggb, x, z0, pw, pb)


def _tc_mlp(poolp, w0, b0, w1, b1, w2, b2, w3, b3):
    def body(pp_r, w0_r, b0_r, w1_r, b1_r, w2_r, b2_r, w3_r, b3_r, o_r):
        p = pp_r[...]
        psum = p[0] + p[1]
        xo = jnp.concatenate(
            [psum[NSUB * k:NSUB * (k + 1)] for k in range(6)], axis=1)
        h = jnp.maximum(jnp.dot(xo, w0_r[...],
                                preferred_element_type=jnp.float32)
                        + b0_r[...], 0.0)
        h = jnp.maximum(jnp.dot(h, w1_r[...],
                                preferred_element_type=jnp.float32)
                        + b1_r[...], 0.0)
        h = jnp.maximum(jnp.dot(h, w2_r[...],
                                preferred_element_type=jnp.float32)
                        + b2_r[...], 0.0)
        o_r[...] = jnp.dot(h, w3_r[...],
                           preferred_element_type=jnp.float32) + b3_r[...]

    return pl.pallas_call(
        body,
        out_shape=jax.ShapeDtypeStruct((NSUB, 4), jnp.float32),
    )(poolp, w0, b0, w1, b1, w2, b2, w3, b3)


def kernel(x, edge_index, batch,
           lin0_W, lin0_b, lin1_W, lin1_b,
           agg0_W, agg0_b, agg1_W, agg1_b,
           cat0_W, cat0_b, cat1_W, cat1_b,
           ex0_W, ex0_b, ex1_W, ex1_b, ex2_W, ex2_b, ex3_W, ex3_b,
           pool0_W, pool0_b, pool1_W, pool1_b, pool2_W, pool2_b):
    f32 = jnp.float32
    row = edge_index[0].astype(jnp.int32)
    col = edge_index[1].astype(jnp.int32)
    npad_e = EPWP - EPW
    prow = (jnp.arange(npad_e, dtype=jnp.int32) * 41) % N
    pcol = N + jnp.arange(npad_e, dtype=jnp.int32) % (NN - N)
    row2 = jnp.concatenate(
        [row.reshape(NW, EPW), jnp.broadcast_to(prow, (NW, npad_e))],
        axis=1).reshape(NW * CPW, ECL)
    col2 = jnp.concatenate(
        [col.reshape(NW, EPW), jnp.broadcast_to(pcol, (NW, npad_e))],
        axis=1).reshape(NW * CPW, ECL)
    zacc = jnp.zeros((NPT, HID), f32)
    zpool = jnp.zeros((6 * NSUB // NS, HID), f32)

    ones = jnp.ones((ECL, HID), f32)
    degp = _sc_degree(ones, col2, zacc)

    h0, hs0 = _tc_stage1(x, degp, lin0_W, lin0_b.reshape(1, HID), agg0_W)
    accp0 = _sc_edge_agg(hs0, row2, col2, zacc)
    z0, h1, hs1 = _tc_stage2_stage1(
        h0, hs0, accp0, degp, cat0_W[:HID], cat0_W[HID:],
        cat0_b.reshape(1, HID), agg0_b.reshape(1, HID),
        lin1_W, lin1_b.reshape(1, HID), agg1_W)
    accp1 = _sc_edge_agg(hs1, row2, col2, zacc)

    pw = jnp.concatenate([pool0_W, pool1_W, pool2_W], axis=1)
    pb = jnp.concatenate([pool0_b, pool1_b, pool2_b]).reshape(1, 3)
    batch_pad = jnp.concatenate(
        [batch.astype(jnp.int32), jnp.zeros((NN - N,), jnp.int32)])
    b6 = batch_pad[None, :] + (jnp.arange(8, dtype=jnp.int32) * NSUB)[:, None]
    v6 = _tc_stage2_values(
        h1, hs1, accp1, degp, cat1_W[:HID], cat1_W[HID:],
        cat1_b.reshape(1, HID), agg1_b.reshape(1, HID), x, z0, pw, pb)
    poolp = _sc_pool(v6, b6, zpool)

    return _tc_mlp(poolp,
                   ex0_W, ex0_b.reshape(1, -1), ex1_W, ex1_b.reshape(1, -1),
                   ex2_W, ex2_b.reshape(1, -1), ex3_W, ex3_b.reshape(1, -1))
